# baseline (speedup 1.0000x reference)
import functools

import jax
import jax.numpy as jnp
from jax import lax
from jax.experimental import pallas as pl
from jax.experimental.pallas import tpu as pltpu
from jax.experimental.pallas import tpu_sc as plsc

VOCAB = 100000
EMBED_DIM = 64
BATCH = 4096
SEQ = 26
SEQ_PAD = 32
NUM_WORKERS = 32
SLABS_PER_WORKER = BATCH // NUM_WORKERS
GROUP = 8
NBUF = 4

_MESH = plsc.VectorSubcoreMesh(core_axis_name="c", subcore_axis_name="s")


@functools.partial(
    pl.kernel,
    out_type=jax.ShapeDtypeStruct((BATCH, SEQ, EMBED_DIM), jnp.float32),
    mesh=_MESH,
    scratch_types=[
        pltpu.VMEM((SLABS_PER_WORKER * SEQ_PAD,), jnp.int32),
        pltpu.VMEM((SLABS_PER_WORKER * SEQ_PAD,), jnp.int32),
        pltpu.VMEM((NBUF, SEQ_PAD, 2 * EMBED_DIM), jnp.float32),
        pltpu.VMEM((2, GROUP, SEQ, EMBED_DIM), jnp.float32),
        pltpu.SemaphoreType.DMA((NBUF,)),
        pltpu.SemaphoreType.DMA((2,)),
    ],
)
def _gather_kernel(table_hbm, idx_hbm, out_hbm, idx_v, pair_v, pbufs, obufs,
                   gsem, wsem):
    wid = lax.axis_index("s") * 2 + lax.axis_index("c")
    sbase = wid * SLABS_PER_WORKER

    n_idx = SLABS_PER_WORKER * SEQ_PAD
    pltpu.sync_copy(idx_hbm.at[pl.ds(wid * n_idx, n_idx)], idx_v)

    def halve(k, _):
        v = idx_v[pl.ds(k * 16, 16)]
        pair_v[pl.ds(k * 16, 16)] = lax.shift_right_logical(v, 1)
        return _
    lax.fori_loop(0, n_idx // 16, halve, None)

    def fire(slab, pb):
        return pltpu.async_copy(
            table_hbm.at[pair_v.at[pl.ds(slab * SEQ_PAD, SEQ_PAD)]],
            pbufs.at[pb], gsem.at[pb])

    for b in range(NBUF):
        fire(b, b)

    def body(slab, _):
        pb = lax.rem(slab, NBUF)
        grp = slab // GROUP
        s = lax.rem(slab, GROUP)
        ob = lax.rem(grp, 2)

        @pl.when(jnp.logical_and(s == 0, grp >= 2))
        def _():
            pltpu.make_async_copy(out_hbm.at[pl.ds(0, GROUP)], obufs.at[ob],
                                  wsem.at[ob]).wait()

        pltpu.make_async_copy(table_hbm.at[pl.ds(0, SEQ_PAD)], pbufs.at[pb],
                              gsem.at[pb]).wait()

        for half, count in ((0, 16), (1, SEQ - 16)):
            v = idx_v[pl.ds(slab * SEQ_PAD + half * 16, 16)]
            off = (v & 1) * EMBED_DIM
            for r in range(count):
                rr = half * 16 + r
                o = off[r]
                for q in range(EMBED_DIM // 16):
                    obufs[ob, s, rr, pl.ds(q * 16, 16)] = (
                        pbufs[pb, rr, pl.ds(o + q * 16, 16)])

        @pl.when(slab + NBUF < SLABS_PER_WORKER)
        def _():
            fire(slab + NBUF, pb)

        @pl.when(s == GROUP - 1)
        def _():
            pltpu.async_copy(
                obufs.at[ob],
                out_hbm.at[pl.ds(sbase + grp * GROUP, GROUP)], wsem.at[ob])
        return _

    lax.fori_loop(0, SLABS_PER_WORKER, body, None)

    for ob in range(2):
        pltpu.make_async_copy(out_hbm.at[pl.ds(0, GROUP)], obufs.at[ob],
                              wsem.at[ob]).wait()


def kernel(indices, table):
    idx = jnp.pad(indices.astype(jnp.int32), ((0, 0), (0, SEQ_PAD - SEQ)))
    tbl2 = table.reshape(VOCAB // 2, 2 * EMBED_DIM)
    return _gather_kernel(tbl2, idx.reshape(-1))
